# baseline (speedup 1.0000x reference)
import functools

import jax
import jax.numpy as jnp
from jax import lax
from jax.experimental import pallas as pl
from jax.experimental.pallas import tpu as pltpu
from jax.experimental.pallas import tpu_sc as plsc

_NUM_ROWS = 64
_ROW_STRIDE = 1543
_D = 512
_NW = 16
_ROWS_PER_W = _NUM_ROWS // _NW


def _make_sc_gather():
    mesh = plsc.VectorSubcoreMesh(
        core_axis_name="c", subcore_axis_name="s", num_cores=1
    )

    @functools.partial(
        pl.kernel,
        mesh=mesh,
        out_type=jax.ShapeDtypeStruct((_NUM_ROWS, _D), jnp.float32),
        scratch_types=[pltpu.SemaphoreType.DMA],
    )
    def sc_gather(table_hbm, out_hbm, sem):
        wid = lax.axis_index("s")
        copies = [
            pltpu.async_copy(
                table_hbm.at[pl.ds((wid * _ROWS_PER_W + j) * _ROW_STRIDE, 1)],
                out_hbm.at[pl.ds(wid * _ROWS_PER_W + j, 1)],
                sem,
            )
            for j in range(_ROWS_PER_W)
        ]
        for c in copies:
            c.wait()

    return sc_gather


_sc_gather = _make_sc_gather()


def kernel(inputs):
    return _sc_gather(inputs)


# speedup vs baseline: 1.1639x; 1.1639x over previous
import functools

import jax
import jax.numpy as jnp
from jax import lax
from jax.experimental import pallas as pl
from jax.experimental.pallas import tpu as pltpu
from jax.experimental.pallas import tpu_sc as plsc

_NUM_ROWS = 64
_ROW_STRIDE = 1543
_D = 512
_L = 16
_NW_ACTIVE = _NUM_ROWS // _L


def _make_sc_gather():
    mesh = plsc.VectorSubcoreMesh(
        core_axis_name="c", subcore_axis_name="s", num_cores=1
    )

    @functools.partial(
        pl.kernel,
        mesh=mesh,
        out_type=jax.ShapeDtypeStruct((_NUM_ROWS, _D), jnp.float32),
        scratch_types=[
            pltpu.VMEM((_L, _D), jnp.float32),
            pltpu.SemaphoreType.DMA,
        ],
    )
    def sc_gather(table_hbm, out_hbm, rows_v, sem):
        wid = lax.axis_index("s")

        @pl.when(wid < _NW_ACTIVE)
        def _():
            idx = (lax.iota(jnp.int32, _L) + wid * _L) * _ROW_STRIDE
            pltpu.async_copy(table_hbm.at[idx], rows_v, sem).wait()
            pltpu.sync_copy(rows_v, out_hbm.at[pl.ds(wid * _L, _L)])

    return sc_gather


_sc_gather = _make_sc_gather()


def kernel(inputs):
    return _sc_gather(inputs)


# speedup vs baseline: 1.1929x; 1.0249x over previous
import functools

import jax
import jax.numpy as jnp
from jax import lax
from jax.experimental import pallas as pl
from jax.experimental.pallas import tpu as pltpu
from jax.experimental.pallas import tpu_sc as plsc

_NUM_ROWS = 64
_ROW_STRIDE = 1543
_D = 512
_L = 16
_RPW = 8
_NW_ACTIVE = _NUM_ROWS // _RPW


def _make_sc_gather():
    mesh = plsc.VectorSubcoreMesh(
        core_axis_name="c", subcore_axis_name="s", num_cores=1
    )

    @functools.partial(
        pl.kernel,
        mesh=mesh,
        out_type=jax.ShapeDtypeStruct((_NUM_ROWS, _D), jnp.float32),
        scratch_types=[
            pltpu.VMEM((_L,), jnp.int32),
            pltpu.VMEM((_RPW, _D), jnp.float32),
            pltpu.SemaphoreType.DMA,
        ],
    )
    def sc_gather(table_hbm, out_hbm, idx_v, rows_v, sem):
        wid = lax.axis_index("s")

        @pl.when(wid < _NW_ACTIVE)
        def _():
            lane = jnp.minimum(lax.iota(jnp.int32, _L), _RPW - 1)
            idx_v[...] = (lane + wid * _RPW) * _ROW_STRIDE
            pltpu.async_copy(
                table_hbm.at[idx_v.at[pl.ds(0, _RPW)]], rows_v, sem
            ).wait()
            pltpu.sync_copy(rows_v, out_hbm.at[pl.ds(wid * _RPW, _RPW)])

    return sc_gather


_sc_gather = _make_sc_gather()


def kernel(inputs):
    return _sc_gather(inputs)


# speedup vs baseline: 1.2159x; 1.0193x over previous
import functools

import jax
import jax.numpy as jnp
from jax import lax
from jax.experimental import pallas as pl
from jax.experimental.pallas import tpu as pltpu
from jax.experimental.pallas import tpu_sc as plsc

_NUM_ROWS = 64
_ROW_STRIDE = 1543
_D = 512
_L = 16
_RPW = 4
_NW_ACTIVE = _NUM_ROWS // _RPW


def _make_sc_gather():
    mesh = plsc.VectorSubcoreMesh(
        core_axis_name="c", subcore_axis_name="s", num_cores=1
    )

    @functools.partial(
        pl.kernel,
        mesh=mesh,
        out_type=jax.ShapeDtypeStruct((_NUM_ROWS, _D), jnp.float32),
        scratch_types=[
            pltpu.VMEM((_L,), jnp.int32),
            pltpu.VMEM((_RPW, _D), jnp.float32),
            pltpu.SemaphoreType.DMA,
        ],
    )
    def sc_gather(table_hbm, out_hbm, idx_v, rows_v, sem):
        wid = lax.axis_index("s")

        @pl.when(wid < _NW_ACTIVE)
        def _():
            lane = jnp.minimum(lax.iota(jnp.int32, _L), _RPW - 1)
            idx_v[...] = (lane + wid * _RPW) * _ROW_STRIDE
            pltpu.async_copy(
                table_hbm.at[idx_v.at[pl.ds(0, _RPW)]], rows_v, sem
            ).wait()
            pltpu.sync_copy(rows_v, out_hbm.at[pl.ds(wid * _RPW, _RPW)])

    return sc_gather


_sc_gather = _make_sc_gather()


def kernel(inputs):
    return _sc_gather(inputs)
